# baseline (speedup 1.0000x reference)
import functools

import jax
import jax.numpy as jnp
import numpy as np
from jax import lax
from jax.experimental import pallas as pl
from jax.experimental.pallas import tpu as pltpu
from jax.experimental.pallas import tpu_sc as plsc

B = 4096
F = 26
D = 16
V = 1000000
NC = 2
NS = 16
NW = NC * NS
TOTAL = B * F
PER_W = TOTAL // NW

LANES = 128
NCOL = V // LANES
NCOL_EVEN = NCOL - (NCOL % NW)
COL_PER_W = NCOL_EVEN // NW
REM_COLS = NCOL - NCOL_EVEN
TAIL = V - NCOL * LANES
VP = V + (LANES - TAIL)
SLANES = 4 * LANES
SCOL_PER_W = COL_PER_W // 4
SPAIRS = SCOL_PER_W // 2
SREM = REM_COLS // 4

BLK = 1024

_SUM_MAT = np.tile(np.eye(D, dtype=np.float32), (F, 1))


@functools.lru_cache(maxsize=1)
def _make_sc_transpose():
    mesh = plsc.VectorSubcoreMesh(
        core_axis_name="c", subcore_axis_name="s", num_cores=NC, num_subcores=NS
    )

    @functools.partial(
        pl.kernel,
        out_type=(jax.ShapeDtypeStruct((VP * D,), jnp.float32),),
        mesh=mesh,
        compiler_params=pltpu.CompilerParams(
            use_tc_tiling_on_sc=True, needs_layout_passes=False
        ),
        scratch_types=[
            pltpu.VMEM((D, SLANES), jnp.float32),
            pltpu.VMEM((D, SLANES), jnp.float32),
            pltpu.VMEM((SLANES * D,), jnp.float32),
            pltpu.SemaphoreType.DMA,
            pltpu.SemaphoreType.DMA,
        ],
    )
    def _sc_transpose(emb_t, tail_flat, out, colA, colB, rowbuf, semA, semB):
        wid = lax.axis_index("s") * NC + lax.axis_index("c")

        row_iota = lax.iota(jnp.int32, D)
        base = wid * SCOL_PER_W

        def start(cs, buf, sem):
            pltpu.async_copy(emb_t.at[:, pl.ds(cs * SLANES, SLANES)], buf, sem)

        def wait(cs, buf, sem):
            pltpu.make_async_copy(
                emb_t.at[:, pl.ds(cs * SLANES, SLANES)], buf, sem).wait()

        scaled_iota = row_iota * D

        def extract(buf, cs):
            @plsc.parallel_loop(0, D, 1, unroll=16)
            def _(d):
                idxv = scaled_iota + d
                for l0 in range(0, SLANES, D):
                    vec = buf[d, pl.ds(l0, D)]
                    plsc.store_scatter(rowbuf, [idxv + l0 * D], vec)

            pltpu.sync_copy(rowbuf, out.at[pl.ds(cs * SLANES * D, SLANES * D)])

        start(base, colA, semA)

        def body(t, _):
            c0 = base + 2 * t
            wait(c0, colA, semA)
            start(c0 + 1, colB, semB)
            extract(colA, c0)

            start(c0 + 2, colA, semA)
            wait(c0 + 1, colB, semB)
            extract(colB, c0 + 1)
            return 0

        lax.fori_loop(0, SPAIRS, body, 0, unroll=False)

        last = base + 2 * SPAIRS
        wait(last, colA, semA)
        extract(colA, last)

        @pl.when(wid < SREM)
        def _():
            cs = NCOL_EVEN // (SLANES // LANES)
            pltpu.sync_copy(emb_t.at[:, pl.ds(cs * SLANES, SLANES)], colA)
            extract(colA, cs)

        @pl.when(wid == REM_COLS)
        def _():
            pltpu.sync_copy(tail_flat, rowbuf.at[pl.ds(0, TAIL * D)])
            pltpu.sync_copy(rowbuf.at[pl.ds(0, TAIL * D)],
                            out.at[pl.ds(NCOL * LANES * D, TAIL * D)])

    return _sc_transpose


@functools.lru_cache(maxsize=1)
def _make_sc_gather():
    mesh = plsc.VectorSubcoreMesh(
        core_axis_name="c", subcore_axis_name="s", num_cores=NC, num_subcores=NS
    )

    @functools.partial(
        pl.kernel,
        out_type=(
            jax.ShapeDtypeStruct((NW, PER_W, D), jnp.float32),
            jax.ShapeDtypeStruct((NW, PER_W), jnp.float32),
        ),
        mesh=mesh,
        compiler_params=pltpu.CompilerParams(use_tc_tiling_on_sc=False),
        scratch_types=[
            pltpu.VMEM((PER_W,), jnp.int32),
            pltpu.VMEM((PER_W, D), jnp.float32),
            pltpu.VMEM((PER_W,), jnp.float32),
            pltpu.SemaphoreType.DMA,
            pltpu.SemaphoreType.DMA,
        ],
    )
    def _sc_gather(x_hbm, emb_hbm, fc_hbm, emb_out, fc_out, idx_v, rows_v,
                   fcv_v, sem_e, sem_f):
        wid = lax.axis_index("s") * NC + lax.axis_index("c")
        pltpu.sync_copy(x_hbm.at[wid], idx_v)
        ce = pltpu.async_copy(emb_hbm.at[idx_v], rows_v, sem_e)
        cf = pltpu.async_copy(fc_hbm.at[idx_v], fcv_v, sem_f)
        ce.wait()
        cf.wait()
        pltpu.sync_copy(rows_v, emb_out.at[wid])
        pltpu.sync_copy(fcv_v, fc_out.at[wid])

    return _sc_gather


def _tc_body(e_ref, fc_ref, a_ref, w0_ref, b0_ref, w1_ref, b1_ref, w2_ref,
             c_ref, out_ref):
    e = e_ref[...]
    a = a_ref[...]
    s = jnp.dot(e, a, preferred_element_type=jnp.float32)
    ss = jnp.dot(e * e, a, preferred_element_type=jnp.float32)
    fm = jnp.sum(fc_ref[...], axis=1) + 0.5 * jnp.sum(s * s - ss, axis=1)
    h = jnp.maximum(jnp.dot(e, w0_ref[...], preferred_element_type=jnp.float32)
                    + b0_ref[...], 0.0)
    h = jnp.maximum(jnp.dot(h, w1_ref[...], preferred_element_type=jnp.float32)
                    + b1_ref[...], 0.0)
    mlp = jnp.dot(h, w2_ref[...], preferred_element_type=jnp.float32)
    z = fm + mlp[:, 0] + c_ref[0]
    out_ref[...] = 1.0 / (1.0 + jnp.exp(-z))


def _tc_dense(e, fcm, w0, b0, w1, b1, w2, const):
    a = jnp.asarray(_SUM_MAT)
    grid = (B // BLK,)
    return pl.pallas_call(
        _tc_body,
        grid=grid,
        in_specs=[
            pl.BlockSpec((BLK, F * D), lambda i: (i, 0)),
            pl.BlockSpec((BLK, F), lambda i: (i, 0)),
            pl.BlockSpec((F * D, D), lambda i: (0, 0)),
            pl.BlockSpec((F * D, 256), lambda i: (0, 0)),
            pl.BlockSpec((256,), lambda i: (0,)),
            pl.BlockSpec((256, 128), lambda i: (0, 0)),
            pl.BlockSpec((128,), lambda i: (0,)),
            pl.BlockSpec((128, 1), lambda i: (0, 0)),
            pl.BlockSpec(memory_space=pltpu.SMEM),
        ],
        out_specs=pl.BlockSpec((BLK,), lambda i: (i,)),
        out_shape=jax.ShapeDtypeStruct((B,), jnp.float32),
    )(e, fcm, a, w0, b0, w1, b1, w2, const)


def kernel(x, emb_table, fc_table, bias, W0, b0, W1, b1, W2, b2):
    emb_t = emb_table.T
    tail = lax.slice(emb_table, (NCOL * LANES, 0), (V, D)).reshape(TAIL * D)
    (tbl_flat,) = _make_sc_transpose()(emb_t, tail)
    tbl2d = tbl_flat.reshape(VP, D)
    xf = x.reshape(NW, PER_W)
    emb_rows, fc_rows = _make_sc_gather()(xf, tbl2d, fc_table.reshape(V))
    e = emb_rows.reshape(B, F * D)
    fcm = fc_rows.reshape(B, F)
    const = bias + b2
    return _tc_dense(e, fcm, W0, b0, W1, b1, W2, const)


# speedup vs baseline: 1.5238x; 1.5238x over previous
import functools

import jax
import jax.numpy as jnp
import numpy as np
from jax import lax
from jax.experimental import pallas as pl
from jax.experimental.pallas import tpu as pltpu
from jax.experimental.pallas import tpu_sc as plsc

B = 4096
F = 26
D = 16
V = 1000000
NC = 2
NS = 16
NW = NC * NS
TOTAL = B * F
PER_W = TOTAL // NW

LANES = 128
NCOL = V // LANES
NCOL_EVEN = NCOL - (NCOL % NW)
COL_PER_W = NCOL_EVEN // NW
REM_COLS = NCOL - NCOL_EVEN
TAIL = V - NCOL * LANES
VP = V + (LANES - TAIL)
SLANES = 4 * LANES
SCOL_PER_W = COL_PER_W // 4
SPAIRS = SCOL_PER_W // 2
SREM = REM_COLS // 4

BLK = 1024

_SUM_MAT = np.tile(np.eye(D, dtype=np.float32), (F, 1))


@functools.lru_cache(maxsize=1)
def _make_sc_transpose():
    mesh = plsc.VectorSubcoreMesh(
        core_axis_name="c", subcore_axis_name="s", num_cores=NC, num_subcores=NS
    )

    @functools.partial(
        pl.kernel,
        out_type=(jax.ShapeDtypeStruct((VP * D,), jnp.float32),),
        mesh=mesh,
        compiler_params=pltpu.CompilerParams(
            use_tc_tiling_on_sc=True, needs_layout_passes=False
        ),
        scratch_types=[
            pltpu.VMEM((D, SLANES), jnp.float32),
            pltpu.VMEM((D, SLANES), jnp.float32),
            pltpu.VMEM((SLANES * D,), jnp.float32),
            pltpu.SemaphoreType.DMA,
            pltpu.SemaphoreType.DMA,
        ],
    )
    def _sc_transpose(emb_t, tail_flat, out, colA, colB, rowbuf, semA, semB):
        wid = lax.axis_index("s") * NC + lax.axis_index("c")

        row_iota = lax.iota(jnp.int32, D)
        base = wid * SCOL_PER_W

        def start(cs, buf, sem):
            pltpu.async_copy(emb_t.at[:, pl.ds(cs * SLANES, SLANES)], buf, sem)

        def wait(cs, buf, sem):
            pltpu.make_async_copy(
                emb_t.at[:, pl.ds(cs * SLANES, SLANES)], buf, sem).wait()

        scaled_iota = row_iota * D

        def extract(buf, cs):
            @plsc.parallel_loop(0, D, 1, unroll=8)
            def _(d):
                idxv = scaled_iota + d
                for l0 in range(0, SLANES, D):
                    vec = buf[d, pl.ds(l0, D)]
                    plsc.store_scatter(rowbuf, [idxv + l0 * D], vec)

            pltpu.sync_copy(rowbuf, out.at[pl.ds(cs * SLANES * D, SLANES * D)])

        start(base, colA, semA)
        start(base + 1, colB, semB)

        def body(t, _):
            c0 = base + 2 * t
            wait(c0, colA, semA)
            extract(colA, c0)
            start(c0 + 2, colA, semA)
            wait(c0 + 1, colB, semB)
            extract(colB, c0 + 1)

            @pl.when(t + 1 < SPAIRS)
            def _():
                start(c0 + 3, colB, semB)

            return 0

        lax.fori_loop(0, SPAIRS, body, 0, unroll=False)

        last = base + 2 * SPAIRS
        wait(last, colA, semA)
        extract(colA, last)

        @pl.when(wid < SREM)
        def _():
            cs = NCOL_EVEN // (SLANES // LANES)
            pltpu.sync_copy(emb_t.at[:, pl.ds(cs * SLANES, SLANES)], colA)
            extract(colA, cs)

        @pl.when(wid == REM_COLS)
        def _():
            pltpu.sync_copy(tail_flat, rowbuf.at[pl.ds(0, TAIL * D)])
            pltpu.sync_copy(rowbuf.at[pl.ds(0, TAIL * D)],
                            out.at[pl.ds(NCOL * LANES * D, TAIL * D)])

    return _sc_transpose


@functools.lru_cache(maxsize=1)
def _make_sc_gather():
    mesh = plsc.VectorSubcoreMesh(
        core_axis_name="c", subcore_axis_name="s", num_cores=NC, num_subcores=NS
    )

    @functools.partial(
        pl.kernel,
        out_type=(
            jax.ShapeDtypeStruct((NW, PER_W, D), jnp.float32),
            jax.ShapeDtypeStruct((NW, PER_W), jnp.float32),
        ),
        mesh=mesh,
        compiler_params=pltpu.CompilerParams(use_tc_tiling_on_sc=False),
        scratch_types=[
            pltpu.VMEM((PER_W,), jnp.int32),
            pltpu.VMEM((PER_W, D), jnp.float32),
            pltpu.VMEM((PER_W,), jnp.float32),
            pltpu.SemaphoreType.DMA,
            pltpu.SemaphoreType.DMA,
        ],
    )
    def _sc_gather(x_hbm, emb_hbm, fc_hbm, emb_out, fc_out, idx_v, rows_v,
                   fcv_v, sem_e, sem_f):
        wid = lax.axis_index("s") * NC + lax.axis_index("c")
        pltpu.sync_copy(x_hbm.at[wid], idx_v)
        ce = pltpu.async_copy(emb_hbm.at[idx_v], rows_v, sem_e)
        cf = pltpu.async_copy(fc_hbm.at[idx_v], fcv_v, sem_f)
        ce.wait()
        cf.wait()
        pltpu.sync_copy(rows_v, emb_out.at[wid])
        pltpu.sync_copy(fcv_v, fc_out.at[wid])

    return _sc_gather


def _tc_body(e_ref, fc_ref, a_ref, w0_ref, b0_ref, w1_ref, b1_ref, w2_ref,
             c_ref, out_ref):
    e = e_ref[...]
    a = a_ref[...]
    s = jnp.dot(e, a, preferred_element_type=jnp.float32)
    ss = jnp.dot(e * e, a, preferred_element_type=jnp.float32)
    fm = jnp.sum(fc_ref[...], axis=1) + 0.5 * jnp.sum(s * s - ss, axis=1)
    h = jnp.maximum(jnp.dot(e, w0_ref[...], preferred_element_type=jnp.float32)
                    + b0_ref[...], 0.0)
    h = jnp.maximum(jnp.dot(h, w1_ref[...], preferred_element_type=jnp.float32)
                    + b1_ref[...], 0.0)
    mlp = jnp.dot(h, w2_ref[...], preferred_element_type=jnp.float32)
    z = fm + mlp[:, 0] + c_ref[0]
    out_ref[...] = 1.0 / (1.0 + jnp.exp(-z))


def _tc_dense(e, fcm, w0, b0, w1, b1, w2, const):
    a = jnp.asarray(_SUM_MAT)
    grid = (B // BLK,)
    return pl.pallas_call(
        _tc_body,
        grid=grid,
        in_specs=[
            pl.BlockSpec((BLK, F * D), lambda i: (i, 0)),
            pl.BlockSpec((BLK, F), lambda i: (i, 0)),
            pl.BlockSpec((F * D, D), lambda i: (0, 0)),
            pl.BlockSpec((F * D, 256), lambda i: (0, 0)),
            pl.BlockSpec((256,), lambda i: (0,)),
            pl.BlockSpec((256, 128), lambda i: (0, 0)),
            pl.BlockSpec((128,), lambda i: (0,)),
            pl.BlockSpec((128, 1), lambda i: (0, 0)),
            pl.BlockSpec(memory_space=pltpu.SMEM),
        ],
        out_specs=pl.BlockSpec((BLK,), lambda i: (i,)),
        out_shape=jax.ShapeDtypeStruct((B,), jnp.float32),
    )(e, fcm, a, w0, b0, w1, b1, w2, const)


def kernel(x, emb_table, fc_table, bias, W0, b0, W1, b1, W2, b2):
    emb_t = emb_table.T
    tail = lax.slice(emb_table, (NCOL * LANES, 0), (V, D)).reshape(TAIL * D)
    (tbl_flat,) = _make_sc_transpose()(emb_t, tail)
    tbl2d = tbl_flat.reshape(VP, D)
    xf = x.reshape(NW, PER_W)
    emb_rows, fc_rows = _make_sc_gather()(xf, tbl2d, fc_table.reshape(V))
    e = emb_rows.reshape(B, F * D)
    fcm = fc_rows.reshape(B, F)
    const = bias + b2
    return _tc_dense(e, fcm, W0, b0, W1, b1, W2, const)


# speedup vs baseline: 1.5423x; 1.0121x over previous
import functools

import jax
import jax.numpy as jnp
import numpy as np
from jax import lax
from jax.experimental import pallas as pl
from jax.experimental.pallas import tpu as pltpu
from jax.experimental.pallas import tpu_sc as plsc

B = 4096
F = 26
D = 16
V = 1000000
NC = 2
NS = 16
NW = NC * NS
TOTAL = B * F
PER_W = TOTAL // NW

LANES = 128
NCOL = V // LANES
NCOL_EVEN = NCOL - (NCOL % NW)
COL_PER_W = NCOL_EVEN // NW
REM_COLS = NCOL - NCOL_EVEN
TAIL = V - NCOL * LANES
VP = V + (LANES - TAIL)
SLANES = 4 * LANES
SCOL_PER_W = COL_PER_W // 4
SPAIRS = SCOL_PER_W // 2
SREM = REM_COLS // 4

BLK = 1024

_SUM_MAT = np.tile(np.eye(D, dtype=np.float32), (F, 1))


@functools.lru_cache(maxsize=1)
def _make_sc_transpose():
    mesh = plsc.VectorSubcoreMesh(
        core_axis_name="c", subcore_axis_name="s", num_cores=NC, num_subcores=NS
    )

    @functools.partial(
        pl.kernel,
        out_type=(jax.ShapeDtypeStruct((VP * D,), jnp.float32),),
        mesh=mesh,
        compiler_params=pltpu.CompilerParams(
            use_tc_tiling_on_sc=True, needs_layout_passes=False
        ),
        scratch_types=[
            pltpu.VMEM((D, SLANES), jnp.float32),
            pltpu.VMEM((D, SLANES), jnp.float32),
            pltpu.VMEM((SLANES * D,), jnp.float32),
            pltpu.SemaphoreType.DMA,
            pltpu.SemaphoreType.DMA,
        ],
    )
    def _sc_transpose(emb_t, tail_flat, out, colA, colB, rowbuf, semA, semB):
        wid = lax.axis_index("s") * NC + lax.axis_index("c")

        row_iota = lax.iota(jnp.int32, D)
        base = wid * SCOL_PER_W

        def start(cs, buf, sem):
            pltpu.async_copy(emb_t.at[:, pl.ds(cs * SLANES, SLANES)], buf, sem)

        def wait(cs, buf, sem):
            pltpu.make_async_copy(
                emb_t.at[:, pl.ds(cs * SLANES, SLANES)], buf, sem).wait()

        scaled_iota = row_iota * D

        def extract(buf, cs):
            @plsc.parallel_loop(0, D, 1, unroll=4)
            def _(d):
                idxv = scaled_iota + d
                for l0 in range(0, SLANES, D):
                    vec = buf[d, pl.ds(l0, D)]
                    plsc.store_scatter(rowbuf, [idxv + l0 * D], vec)

            pltpu.sync_copy(rowbuf, out.at[pl.ds(cs * SLANES * D, SLANES * D)])

        start(base, colA, semA)
        start(base + 1, colB, semB)

        def body(t, _):
            c0 = base + 2 * t
            wait(c0, colA, semA)
            extract(colA, c0)
            start(c0 + 2, colA, semA)
            wait(c0 + 1, colB, semB)
            extract(colB, c0 + 1)

            @pl.when(t + 1 < SPAIRS)
            def _():
                start(c0 + 3, colB, semB)

            return 0

        lax.fori_loop(0, SPAIRS, body, 0, unroll=False)

        last = base + 2 * SPAIRS
        wait(last, colA, semA)
        extract(colA, last)

        @pl.when(wid < SREM)
        def _():
            cs = NCOL_EVEN // (SLANES // LANES)
            pltpu.sync_copy(emb_t.at[:, pl.ds(cs * SLANES, SLANES)], colA)
            extract(colA, cs)

        @pl.when(wid == REM_COLS)
        def _():
            pltpu.sync_copy(tail_flat, rowbuf.at[pl.ds(0, TAIL * D)])
            pltpu.sync_copy(rowbuf.at[pl.ds(0, TAIL * D)],
                            out.at[pl.ds(NCOL * LANES * D, TAIL * D)])

    return _sc_transpose


@functools.lru_cache(maxsize=1)
def _make_sc_gather():
    mesh = plsc.VectorSubcoreMesh(
        core_axis_name="c", subcore_axis_name="s", num_cores=NC, num_subcores=NS
    )

    @functools.partial(
        pl.kernel,
        out_type=(
            jax.ShapeDtypeStruct((NW, PER_W, D), jnp.float32),
            jax.ShapeDtypeStruct((NW, PER_W), jnp.float32),
        ),
        mesh=mesh,
        compiler_params=pltpu.CompilerParams(use_tc_tiling_on_sc=False),
        scratch_types=[
            pltpu.VMEM((PER_W,), jnp.int32),
            pltpu.VMEM((PER_W, D), jnp.float32),
            pltpu.VMEM((PER_W,), jnp.float32),
            pltpu.SemaphoreType.DMA,
            pltpu.SemaphoreType.DMA,
        ],
    )
    def _sc_gather(x_hbm, emb_hbm, fc_hbm, emb_out, fc_out, idx_v, rows_v,
                   fcv_v, sem_e, sem_f):
        wid = lax.axis_index("s") * NC + lax.axis_index("c")
        pltpu.sync_copy(x_hbm.at[wid], idx_v)
        ce = pltpu.async_copy(emb_hbm.at[idx_v], rows_v, sem_e)
        cf = pltpu.async_copy(fc_hbm.at[idx_v], fcv_v, sem_f)
        ce.wait()
        cf.wait()
        pltpu.sync_copy(rows_v, emb_out.at[wid])
        pltpu.sync_copy(fcv_v, fc_out.at[wid])

    return _sc_gather


def _tc_body(e_ref, fc_ref, a_ref, w0_ref, b0_ref, w1_ref, b1_ref, w2_ref,
             c_ref, out_ref):
    e = e_ref[...]
    a = a_ref[...]
    s = jnp.dot(e, a, preferred_element_type=jnp.float32)
    ss = jnp.dot(e * e, a, preferred_element_type=jnp.float32)
    fm = jnp.sum(fc_ref[...], axis=1) + 0.5 * jnp.sum(s * s - ss, axis=1)
    h = jnp.maximum(jnp.dot(e, w0_ref[...], preferred_element_type=jnp.float32)
                    + b0_ref[...], 0.0)
    h = jnp.maximum(jnp.dot(h, w1_ref[...], preferred_element_type=jnp.float32)
                    + b1_ref[...], 0.0)
    mlp = jnp.dot(h, w2_ref[...], preferred_element_type=jnp.float32)
    z = fm + mlp[:, 0] + c_ref[0]
    out_ref[...] = 1.0 / (1.0 + jnp.exp(-z))


def _tc_dense(e, fcm, w0, b0, w1, b1, w2, const):
    a = jnp.asarray(_SUM_MAT)
    grid = (B // BLK,)
    return pl.pallas_call(
        _tc_body,
        grid=grid,
        in_specs=[
            pl.BlockSpec((BLK, F * D), lambda i: (i, 0)),
            pl.BlockSpec((BLK, F), lambda i: (i, 0)),
            pl.BlockSpec((F * D, D), lambda i: (0, 0)),
            pl.BlockSpec((F * D, 256), lambda i: (0, 0)),
            pl.BlockSpec((256,), lambda i: (0,)),
            pl.BlockSpec((256, 128), lambda i: (0, 0)),
            pl.BlockSpec((128,), lambda i: (0,)),
            pl.BlockSpec((128, 1), lambda i: (0, 0)),
            pl.BlockSpec(memory_space=pltpu.SMEM),
        ],
        out_specs=pl.BlockSpec((BLK,), lambda i: (i,)),
        out_shape=jax.ShapeDtypeStruct((B,), jnp.float32),
    )(e, fcm, a, w0, b0, w1, b1, w2, const)


def kernel(x, emb_table, fc_table, bias, W0, b0, W1, b1, W2, b2):
    emb_t = emb_table.T
    tail = lax.slice(emb_table, (NCOL * LANES, 0), (V, D)).reshape(TAIL * D)
    (tbl_flat,) = _make_sc_transpose()(emb_t, tail)
    tbl2d = tbl_flat.reshape(VP, D)
    xf = x.reshape(NW, PER_W)
    emb_rows, fc_rows = _make_sc_gather()(xf, tbl2d, fc_table.reshape(V))
    e = emb_rows.reshape(B, F * D)
    fcm = fc_rows.reshape(B, F)
    const = bias + b2
    return _tc_dense(e, fcm, W0, b0, W1, b1, W2, const)
